# baseline (speedup 1.0000x reference)
import functools

import jax
import jax.numpy as jnp
from jax import lax
from jax.experimental import pallas as pl
from jax.experimental.pallas import tpu as pltpu
from jax.experimental.pallas import tpu_sc as plsc

BATCH = 16384
FACTOR = 16
MLP_DIM = 64

_NC = 2
_NS = 16
_NW = _NC * _NS
_BPW = BATCH // _NW
_CHUNK = 128
_NCHUNK = _BPW // _CHUNK


def _sc_gather(user, item, tT_ug, tT_ig, tT_um, tT_im):
    mesh = plsc.VectorSubcoreMesh(core_axis_name="c", subcore_axis_name="s")

    @functools.partial(
        pl.kernel,
        out_type=[
            jax.ShapeDtypeStruct((FACTOR, BATCH), jnp.float32),
            jax.ShapeDtypeStruct((FACTOR, BATCH), jnp.float32),
            jax.ShapeDtypeStruct((MLP_DIM, BATCH), jnp.float32),
            jax.ShapeDtypeStruct((MLP_DIM, BATCH), jnp.float32),
        ],
        mesh=mesh,
        compiler_params=pltpu.CompilerParams(use_tc_tiling_on_sc=False),
        scratch_types=[
            pltpu.VMEM((_BPW,), jnp.int32),
            pltpu.VMEM((_BPW,), jnp.int32),
            pltpu.VMEM((FACTOR, _CHUNK), jnp.float32),
            pltpu.VMEM((FACTOR, _CHUNK), jnp.float32),
            pltpu.VMEM((MLP_DIM, _CHUNK), jnp.float32),
            pltpu.VMEM((MLP_DIM, _CHUNK), jnp.float32),
            pltpu.SemaphoreType.DMA,
        ],
    )
    def k(user_h, item_h, tug_h, tig_h, tum_h, tim_h,
          oug_h, oig_h, oum_h, oim_h,
          uix_v, iix_v, bug, big, bum, bim, sem):
        wid = lax.axis_index("s") * _NC + lax.axis_index("c")
        base = wid * _BPW
        pltpu.sync_copy(user_h.at[pl.ds(base, _BPW)], uix_v)
        pltpu.sync_copy(item_h.at[pl.ds(base, _BPW)], iix_v)
        for c in range(_NCHUNK):
            sl = pl.ds(c * _CHUNK, _CHUNK)
            copies = []
            for f in range(FACTOR):
                copies.append(pltpu.async_copy(
                    tug_h.at[f].at[uix_v.at[sl]], bug.at[f], sem))
                copies.append(pltpu.async_copy(
                    tig_h.at[f].at[iix_v.at[sl]], big.at[f], sem))
            for f in range(MLP_DIM):
                copies.append(pltpu.async_copy(
                    tum_h.at[f].at[uix_v.at[sl]], bum.at[f], sem))
                copies.append(pltpu.async_copy(
                    tim_h.at[f].at[iix_v.at[sl]], bim.at[f], sem))
            for cp in copies:
                cp.wait()
            osl = pl.ds(base + c * _CHUNK, _CHUNK)
            pltpu.sync_copy(bug, oug_h.at[:, osl])
            pltpu.sync_copy(big, oig_h.at[:, osl])
            pltpu.sync_copy(bum, oum_h.at[:, osl])
            pltpu.sync_copy(bim, oim_h.at[:, osl])

    return k(user, item, tT_ug, tT_ig, tT_um, tT_im)


_BB = 2048


def _tc_body(ug_ref, ig_ref, um_ref, im_ref, w0aT_ref, w0bT_ref, b0_ref,
             w1T_ref, b1_ref, w2T_ref, b2_ref, wpg_ref, wph_ref, bp_ref,
             out_ref):
    gmf = ug_ref[...] * ig_ref[...]
    h = w0aT_ref[...] @ um_ref[...] + w0bT_ref[...] @ im_ref[...] + b0_ref[...]
    h = jnp.maximum(h, 0.0)
    h = jnp.maximum(w1T_ref[...] @ h + b1_ref[...], 0.0)
    h = jnp.maximum(w2T_ref[...] @ h + b2_ref[...], 0.0)
    acc = jnp.sum(gmf * wpg_ref[...], axis=0) + jnp.sum(h * wph_ref[...], axis=0)
    out_ref[...] = acc + bp_ref[0]


def _tc_dense(ugT, igT, umT, imT, W0, b0, W1, b1, W2, b2, Wp, bp):
    grid = (BATCH // _BB,)

    def col_blk(shape):
        return pl.BlockSpec((shape[0], _BB), lambda i: (0, i))

    def full_blk(shape):
        return pl.BlockSpec(shape, lambda i: (0,) * len(shape))

    w0aT = W0[:MLP_DIM].T
    w0bT = W0[MLP_DIM:].T
    w1T, w2T = W1.T, W2.T
    b0c, b1c, b2c = b0.reshape(-1, 1), b1.reshape(-1, 1), b2.reshape(-1, 1)
    wpg = Wp[:FACTOR].reshape(-1, 1)
    wph = Wp[FACTOR:].reshape(-1, 1)
    in_specs = [
        col_blk(ugT.shape), col_blk(igT.shape),
        col_blk(umT.shape), col_blk(imT.shape),
        full_blk(w0aT.shape), full_blk(w0bT.shape), full_blk(b0c.shape),
        full_blk(w1T.shape), full_blk(b1c.shape),
        full_blk(w2T.shape), full_blk(b2c.shape),
        full_blk(wpg.shape), full_blk(wph.shape), full_blk(bp.shape),
    ]
    return pl.pallas_call(
        _tc_body,
        grid=grid,
        in_specs=in_specs,
        out_specs=pl.BlockSpec((_BB,), lambda i: (i,)),
        out_shape=jax.ShapeDtypeStruct((BATCH,), jnp.float32),
    )(ugT, igT, umT, imT, w0aT, w0bT, b0c, w1T, b1c, w2T, b2c, wpg, wph, bp)


def kernel(user, item, user_emb_gmf, item_emb_gmf, user_emb_mlp, item_emb_mlp,
           W0, b0, W1, b1, W2, b2, Wp, bp):
    user = user.astype(jnp.int32)
    item = item.astype(jnp.int32)
    ugT, igT, umT, imT = _sc_gather(
        user, item, user_emb_gmf.T, item_emb_gmf.T,
        user_emb_mlp.T, item_emb_mlp.T)
    return _tc_dense(ugT, igT, umT, imT, W0, b0, W1, b1, W2, b2, Wp, bp)
